# baseline (speedup 1.0000x reference)
import functools

import jax
import jax.numpy as jnp
from jax import lax
from jax.experimental import pallas as pl
from jax.experimental.pallas import tpu as pltpu
from jax.experimental.pallas import tpu_sc as plsc

B = 1024
SEQ = 77
MAIN = 72
TAIL = SEQ - MAIN
VOCAB = 49408
DIM = 768
LANES = 16
NC = 2
NS = 16
NW = NC * NS
EPW = B // NW
GRP = 8
D_CHUNKS = DIM // LANES

_mesh = plsc.VectorSubcoreMesh(core_axis_name="c", subcore_axis_name="s")


@functools.partial(
    pl.kernel,
    out_type=jax.ShapeDtypeStruct((B, SEQ, DIM), jnp.float32),
    mesh=_mesh,
    scratch_types=[
        pltpu.VMEM((GRP, MAIN), jnp.int32),
        pltpu.VMEM((EPW * 8,), jnp.int32),
        pltpu.VMEM((2, SEQ, DIM), jnp.float32),
        pltpu.VMEM((8, DIM), jnp.float32),
        pltpu.SemaphoreType.DMA((2,)),
        pltpu.SemaphoreType.DMA,
        pltpu.SemaphoreType.DMA((2,)),
    ],
    compiler_params=pltpu.CompilerParams(needs_layout_passes=False),
)
def _emb_kernel(x_hbm, xt_hbm, flag_hbm, table_hbm, pos_hbm, out_hbm,
                idx_v, idxt_v, bufs, tail, gsem, tsem, ssem):
    wid = lax.axis_index("s") * NC + lax.axis_index("c")
    eb = wid * EPW
    pltpu.sync_copy(xt_hbm.at[wid], idxt_v)
    pltpu.sync_copy(flag_hbm, tail.at[0, pl.ds(0, LANES)])
    pos_nonzero = tail[0, pl.ds(0, LANES)][0] != 0.0

    def refill(m):
        pltpu.sync_copy(x_hbm.at[pl.ds(eb + GRP * m, GRP)], idx_v)

    def gather_main(e, p):
        return pltpu.make_async_copy(
            table_hbm.at[idx_v.at[lax.rem(e, GRP)]],
            bufs.at[p, pl.ds(0, MAIN)], gsem.at[p])

    def gather_tail(e):
        return pltpu.make_async_copy(
            table_hbm.at[idxt_v.at[pl.ds(e * 8, 8)]], tail, tsem)

    def store_elem(e, p):
        return pltpu.make_async_copy(
            bufs.at[p], out_hbm.at[eb + e], ssem.at[p])

    refill(0)
    gather_main(0, 0).start()
    gather_tail(0).start()

    def elem_body(e, carry):
        p = lax.rem(e, 2)

        gather_main(e, p).wait()
        gather_tail(e).wait()

        def rep_body(j, cc):
            for d in range(D_CHUNKS):
                sl = pl.ds(d * LANES, LANES)
                bufs[p, MAIN + j, sl] = tail[j, sl]
            return cc

        lax.fori_loop(0, TAIL, rep_body, 0)

        @pl.when(pos_nonzero)
        def _add():
            for g in range(10):
                rows = min(8, SEQ - 8 * g)
                pltpu.async_copy(pos_hbm.at[pl.ds(8 * g, 8)], tail, tsem
                                 ).wait()

                def row_body(j, cc):
                    for d in range(D_CHUNKS):
                        sl = pl.ds(d * LANES, LANES)
                        plsc.addupdate(bufs.at[p, 8 * g + j, sl],
                                       tail[j, sl])
                    return cc

                lax.fori_loop(0, rows, row_body, 0)

        store_elem(e, p).start()

        @pl.when(e + 1 < EPW)
        def _next_tail():
            gather_tail(e + 1).start()

        @pl.when((lax.rem(e, GRP) == GRP - 1) & (e + 1 < EPW))
        def _refill():
            refill((e + 1) // GRP)

        @pl.when(e >= 1)
        def _drain_prev():
            store_elem(e - 1, 1 - p).wait()

        @pl.when(e + 1 < EPW)
        def _next_main():
            gather_main(e + 1, 1 - p).start()

        return carry

    lax.fori_loop(0, EPW, elem_body, 0)
    store_elem(EPW - 1, lax.rem(EPW - 1, 2)).wait()


def kernel(x, token_embedding, pos_embed):
    x2 = x.reshape(B, SEQ).astype(jnp.int32)
    xm = x2[:, :MAIN]
    xt = jnp.pad(x2[:, MAIN:SEQ], ((0, 0), (0, 8 - TAIL))).reshape(NW, EPW * 8)
    flag = jnp.full((LANES,), jnp.any(pos_embed != 0), jnp.float32)
    pos80 = jnp.pad(pos_embed, ((0, 80 - SEQ), (0, 0)))
    return _emb_kernel(xm, xt, flag, token_embedding, pos80)


# speedup vs baseline: 1.0124x; 1.0124x over previous
import functools

import jax
import jax.numpy as jnp
from jax import lax
from jax.experimental import pallas as pl
from jax.experimental.pallas import tpu as pltpu
from jax.experimental.pallas import tpu_sc as plsc

B = 1024
SEQ = 77
VOCAB = 49408
DIM = 768
LANES = 16
NC = 2
NS = 16
NW = NC * NS
ROWS = B * SEQ
ROWS_PER_W = ROWS // NW
K = 32
CHUNKS = ROWS_PER_W // K
D_CHUNKS = DIM // LANES

_mesh = plsc.VectorSubcoreMesh(core_axis_name="c", subcore_axis_name="s")


@functools.partial(
    pl.kernel,
    out_type=jax.ShapeDtypeStruct((ROWS, DIM), jnp.float32),
    mesh=_mesh,
    scratch_types=[
        pltpu.VMEM((ROWS_PER_W,), jnp.int32),
        pltpu.VMEM((SEQ, DIM), jnp.float32),
        pltpu.VMEM((2, K, DIM), jnp.float32),
        pltpu.SemaphoreType.DMA((2,)),
        pltpu.SemaphoreType.DMA((2,)),
    ],
    compiler_params=pltpu.CompilerParams(needs_layout_passes=False),
)
def _emb_kernel(x_hbm, table_hbm, pos_hbm, out_hbm, idx_v, pos_v, bufs,
                gsem, ssem):
    wid = lax.axis_index("s") * NC + lax.axis_index("c")
    base = wid * ROWS_PER_W
    pltpu.sync_copy(x_hbm.at[wid], idx_v)
    pltpu.sync_copy(pos_hbm, pos_v)

    def or_body(i, acc):
        return acc | plsc.bitcast(pos_v[i // D_CHUNKS,
                                        pl.ds((i % D_CHUNKS) * LANES, LANES)],
                                  jnp.int32)

    acc = lax.fori_loop(0, SEQ * D_CHUNKS, or_body,
                        jnp.zeros((LANES,), jnp.int32))
    nzvec = jnp.where(acc != 0, jnp.int32(1), jnp.int32(0))
    pos_nonzero = lax.reduce_max(nzvec, axes=(0,)) > 0

    def gather_chunk(c, p):
        return pltpu.make_async_copy(
            table_hbm.at[idx_v.at[pl.ds(c * K, K)]], bufs.at[p], gsem.at[p])

    def store_chunk(c, p):
        return pltpu.make_async_copy(
            bufs.at[p], out_hbm.at[pl.ds(base + c * K, K)], ssem.at[p])

    gather_chunk(0, 0).start()

    def chunk_body(c, carry):
        p = lax.rem(c, 2)
        gather_chunk(c, p).wait()

        @pl.when(pos_nonzero)
        def _add():
            phase = lax.rem(c * K, SEQ)

            def row_body(j, cc):
                s0 = phase + j
                s = lax.select(s0 >= SEQ, s0 - SEQ, s0)
                for d in range(D_CHUNKS):
                    sl = pl.ds(d * LANES, LANES)
                    plsc.addupdate(bufs.at[p, j, sl], pos_v[s, sl])
                return cc

            lax.fori_loop(0, K, row_body, 0)

        store_chunk(c, p).start()

        @pl.when(c >= 1)
        def _drain_prev():
            store_chunk(c - 1, 1 - p).wait()

        @pl.when(c + 1 < CHUNKS)
        def _next():
            gather_chunk(c + 1, 1 - p).start()

        return carry

    lax.fori_loop(0, CHUNKS, chunk_body, 0)
    store_chunk(CHUNKS - 1, lax.rem(CHUNKS - 1, 2)).wait()


def kernel(x, token_embedding, pos_embed):
    xw = x.reshape(NW, ROWS_PER_W).astype(jnp.int32)
    out = _emb_kernel(xw, token_embedding, pos_embed)
    return out.reshape(B, SEQ, DIM)


# speedup vs baseline: 1.0358x; 1.0231x over previous
import functools

import jax
import jax.numpy as jnp
from jax import lax
from jax.experimental import pallas as pl
from jax.experimental.pallas import tpu as pltpu
from jax.experimental.pallas import tpu_sc as plsc

B = 1024
SEQ = 77
VOCAB = 49408
DIM = 768
LANES = 16
NC = 2
NS = 16
NW = NC * NS
ROWS = B * SEQ
ROWS_PER_W = ROWS // NW
K = 32
CHUNKS = ROWS_PER_W // K
D_CHUNKS = DIM // LANES

_mesh = plsc.VectorSubcoreMesh(core_axis_name="c", subcore_axis_name="s")


@functools.partial(
    pl.kernel,
    out_type=jax.ShapeDtypeStruct((ROWS, DIM), jnp.float32),
    mesh=_mesh,
    scratch_types=[
        pltpu.VMEM((ROWS_PER_W,), jnp.int32),
        pltpu.VMEM((SEQ, DIM), jnp.float32),
        pltpu.VMEM((2, K, DIM), jnp.float32),
        pltpu.SemaphoreType.DMA((2,)),
        pltpu.SemaphoreType.DMA((2,)),
    ],
    compiler_params=pltpu.CompilerParams(needs_layout_passes=False),
)
def _emb_kernel(x_hbm, table_hbm, pos_hbm, out_hbm, idx_v, pos_v, bufs,
                gsem, ssem):
    wid = lax.axis_index("s") * NC + lax.axis_index("c")
    base = wid * ROWS_PER_W
    pltpu.sync_copy(x_hbm.at[wid], idx_v)
    pltpu.sync_copy(pos_hbm, pos_v)

    def or_body(i, acc):
        return acc | plsc.bitcast(pos_v[i // D_CHUNKS,
                                        pl.ds((i % D_CHUNKS) * LANES, LANES)],
                                  jnp.int32)

    acc = lax.fori_loop(0, SEQ * D_CHUNKS, or_body,
                        jnp.zeros((LANES,), jnp.int32))
    nzvec = jnp.where(acc != 0, jnp.int32(1), jnp.int32(0))
    pos_nonzero = lax.reduce_max(nzvec, axes=(0,)) > 0

    def gather_chunk(c, p):
        return pltpu.make_async_copy(
            table_hbm.at[idx_v.at[pl.ds(c * K, K)]], bufs.at[p], gsem.at[p])

    def store_chunk(c, p):
        return pltpu.make_async_copy(
            bufs.at[p], out_hbm.at[pl.ds(base + c * K, K)], ssem.at[p])

    gather_chunk(0, 0).start()
    gather_chunk(1, 1).start()

    def chunk_body(c, carry):
        p = lax.rem(c, 2)
        gather_chunk(c, p).wait()

        @pl.when(pos_nonzero)
        def _add():
            phase = lax.rem(c * K, SEQ)

            def row_body(j, cc):
                s0 = phase + j
                s = lax.select(s0 >= SEQ, s0 - SEQ, s0)
                for d in range(D_CHUNKS):
                    sl = pl.ds(d * LANES, LANES)
                    plsc.addupdate(bufs.at[p, j, sl], pos_v[s, sl])
                return cc

            lax.fori_loop(0, K, row_body, 0)

        store_chunk(c, p).start()
        store_chunk(c, p).wait()

        @pl.when(c + 2 < CHUNKS)
        def _next():
            gather_chunk(c + 2, p).start()

        return carry

    lax.fori_loop(0, CHUNKS, chunk_body, 0)


def kernel(x, token_embedding, pos_embed):
    xw = x.reshape(NW, ROWS_PER_W).astype(jnp.int32)
    out = _emb_kernel(xw, token_embedding, pos_embed)
    return out.reshape(B, SEQ, DIM)
